# Initial kernel scaffold; baseline (speedup 1.0000x reference)
import jax
import jax.numpy as jnp
from jax.experimental import pallas as pl


def kernel(x, edge_index, edge_type, edge_weight, issuer_index, params):
    raise NotImplementedError("write your pallas kernel here")


# speedup vs baseline: 6.3811x; 6.3811x over previous
import functools

import jax
import jax.numpy as jnp
from jax.experimental import pallas as pl
from jax.experimental.pallas import tpu as pltpu

_N = 10000
_E = 320000
_D = 256
_H = 4
_O = 64
_EPS = 1e-5
_NP = 10240
_BR = 512


def _proj_in_body(x_ref, iss_ref, wx_ref, wi_ref, c_ref, o_ref):
    acc = jnp.dot(x_ref[...], wx_ref[...], preferred_element_type=jnp.float32)
    acc += jnp.dot(iss_ref[...], wi_ref[...], preferred_element_type=jnp.float32)
    o_ref[...] = jnp.maximum(acc + c_ref[...], 0.0)


def _proj_in(xp, issp, wx, wi, c):
    grid = (_NP // _BR,)
    return pl.pallas_call(
        _proj_in_body,
        grid=grid,
        in_specs=[
            pl.BlockSpec((_BR, 128), lambda i: (i, 0)),
            pl.BlockSpec((_BR, 16), lambda i: (i, 0)),
            pl.BlockSpec((128, _D), lambda i: (0, 0)),
            pl.BlockSpec((16, _D), lambda i: (0, 0)),
            pl.BlockSpec((1, _D), lambda i: (0, 0)),
        ],
        out_specs=pl.BlockSpec((_BR, _D), lambda i: (i, 0)),
        out_shape=jax.ShapeDtypeStruct((_NP, _D), jnp.float32),
    )(xp, issp, wx, wi, c)


def _lr_body(h_ref, wl_ref, wr_ref, bl_ref, br_ref, xl_ref, xr_ref):
    h = h_ref[...]
    xl_ref[...] = jnp.dot(h, wl_ref[...], preferred_element_type=jnp.float32) + bl_ref[...]
    xr_ref[...] = jnp.dot(h, wr_ref[...], preferred_element_type=jnp.float32) + br_ref[...]


def _proj_lr(h, wl, wr, bl, br):
    grid = (_NP // _BR,)
    return pl.pallas_call(
        _lr_body,
        grid=grid,
        in_specs=[
            pl.BlockSpec((_BR, _D), lambda i: (i, 0)),
            pl.BlockSpec((_D, _D), lambda i: (0, 0)),
            pl.BlockSpec((_D, _D), lambda i: (0, 0)),
            pl.BlockSpec((1, _D), lambda i: (0, 0)),
            pl.BlockSpec((1, _D), lambda i: (0, 0)),
        ],
        out_specs=[
            pl.BlockSpec((_BR, _D), lambda i: (i, 0)),
            pl.BlockSpec((_BR, _D), lambda i: (i, 0)),
        ],
        out_shape=[
            jax.ShapeDtypeStruct((_NP, _D), jnp.float32),
            jax.ShapeDtypeStruct((_NP, _D), jnp.float32),
        ],
    )(h, wl, wr, bl, br)


def _affine_relu_body(a_ref, s_ref, c_ref, o_ref):
    o_ref[...] = jnp.maximum(a_ref[...] * s_ref[...] + c_ref[...], 0.0)


def _affine_relu(agg, s, c):
    grid = (_NP // _BR,)
    return pl.pallas_call(
        _affine_relu_body,
        grid=grid,
        in_specs=[
            pl.BlockSpec((_BR, _D), lambda i: (i, 0)),
            pl.BlockSpec((1, _D), lambda i: (0, 0)),
            pl.BlockSpec((1, _D), lambda i: (0, 0)),
        ],
        out_specs=pl.BlockSpec((_BR, _D), lambda i: (i, 0)),
        out_shape=jax.ShapeDtypeStruct((_NP, _D), jnp.float32),
    )(agg, s, c)


def _gat_layer(h, src, dst, edge_type, wgain, cp):
    xl, xr = _proj_lr(h, cp['Wl'], cp['Wr'], cp['bl'][None, :], cp['br'][None, :])
    r8 = cp['rel_r8']
    u = cp['We'][16]
    e = r8[edge_type] + wgain[:, None] * u[None, :]
    m = xl[src] + xr[dst] + e
    m = jnp.where(m >= 0, m, 0.2 * m).reshape(_E, _H, _O)
    logits = jnp.einsum('eho,ho->eh', m, cp['att'])
    ex = jnp.exp(logits)
    den = jax.ops.segment_sum(ex, dst, num_segments=_N)
    alpha = ex / den[dst]
    msg = xl[src].reshape(_E, _H, _O) * alpha[..., None]
    agg = jax.ops.segment_sum(msg.reshape(_E, _D), dst, num_segments=_N)
    agg = jnp.pad(agg + cp['bias'][None, :], ((0, _NP - _N), (0, 0)))
    return agg


def kernel(x, edge_index, edge_type, edge_weight, issuer_index, params):
    p = params
    src = edge_index[0]
    dst = edge_index[1]
    iss_idx = jnp.clip(issuer_index + 1, 0, 1000)
    iss = p['issuer_emb'][iss_idx]

    sc = 1.0 / jnp.sqrt(1.0 + _EPS)
    s_in = p['g_in'] * sc
    c_in = p['b_in'] * s_in + p['beta_in']
    xp = jnp.pad(x, ((0, _NP - _N), (0, 0)))
    issp = jnp.pad(iss, ((0, _NP - _N), (0, 0)))
    h = _proj_in(xp, issp, p['W_in'][:128] * s_in[None, :],
                 p['W_in'][128:] * s_in[None, :], c_in[None, :])

    for li in ('1', '2'):
        cp = dict(p['conv' + li])
        cp['rel_r8'] = p['rel_emb'] @ cp['We'][:16]
        gain8 = jnp.exp(p['rel_log_gain'])
        wgain = edge_weight * gain8[edge_type]
        agg = _gat_layer(h, src, dst, edge_type, wgain, cp)
        s_l = p['g' + li] * sc
        c_l = p['beta' + li]
        h = _affine_relu(agg, s_l[None, :], c_l[None, :])
    return h[:_N]
